# Initial kernel scaffold; baseline (speedup 1.0000x reference)
import jax
import jax.numpy as jnp
from jax.experimental import pallas as pl


def kernel(features, edge_index, W1, b1, W2, b2):
    raise NotImplementedError("write your pallas kernel here")


# speedup vs baseline: 7.7258x; 7.7258x over previous
import jax
import jax.numpy as jnp
from jax import lax
from jax.experimental import pallas as pl
from jax.experimental.pallas import tpu as pltpu
from jax.experimental.pallas import tpu_sc as plsc

_N = 10000
_D = 128
_E = 320000
_NC, _NS = 2, 16
_NW = _NC * _NS
_CHUNK = 128
_CPW = 80
_NPAD = _CPW * _CHUNK
_EPAD = _NW * _CPW * _CHUNK
_RPT = _NPAD // _NS
_DEG_ROWS = 2 * _EPAD // _CHUNK
_DEG_RPT = _DEG_ROWS // _NW

_sc_mesh = plsc.VectorSubcoreMesh(core_axis_name="c", subcore_axis_name="s")


def _deg_body(em, zn, out, idx, ones_v, acc):
    c = lax.axis_index("c")
    s = lax.axis_index("s")
    base = (c * _NS + s) * _DEG_RPT
    pltpu.sync_copy(zn.at[pl.ds(s * _RPT, _RPT)], acc.at[pl.ds(s * _RPT, _RPT)])
    pltpu.sync_copy(em.at[pl.ds(base, _DEG_RPT), :], idx)
    for i in range(_CHUNK // 16):
        ones_v[pl.ds(i * 16, 16)] = jnp.full((16,), 1.0, jnp.float32)
    plsc.subcore_barrier()

    def step(j, carry):
        pltpu.sync_copy(ones_v, acc.at[idx.at[j]], add=True)
        return carry

    lax.fori_loop(0, _DEG_RPT, step, 0)
    plsc.subcore_barrier()
    pltpu.sync_copy(acc.at[pl.ds(s * _RPT, _RPT)],
                    out.at[pl.ds(c * _NPAD + s * _RPT, _RPT)])


_deg_kernel = pl.kernel(
    _deg_body,
    out_type=jax.ShapeDtypeStruct((2 * _NPAD,), jnp.float32),
    mesh=_sc_mesh,
    scratch_types=[
        pltpu.VMEM((_DEG_RPT, _CHUNK), jnp.int32),
        pltpu.VMEM((_CHUNK,), jnp.float32),
        pltpu.VMEM_SHARED((_NPAD,), jnp.float32),
    ],
)


def _mp_body(hs, srcm, dstm, zz, out, idx_s, idx_d, rows, acc, sem):
    c = lax.axis_index("c")
    s = lax.axis_index("s")
    base = (c * _NS + s) * _CPW
    pltpu.sync_copy(zz.at[pl.ds(s * _RPT, _RPT), :],
                    acc.at[pl.ds(s * _RPT, _RPT), :])
    pltpu.sync_copy(srcm.at[pl.ds(base, _CPW), :], idx_s)
    pltpu.sync_copy(dstm.at[pl.ds(base, _CPW), :], idx_d)
    plsc.subcore_barrier()

    def step(j, carry):
        pltpu.async_copy(hs.at[idx_s.at[j]], rows, sem).wait()
        pltpu.sync_copy(rows, acc.at[idx_d.at[j]], add=True)
        return carry

    lax.fori_loop(0, _CPW, step, 0)
    plsc.subcore_barrier()
    pltpu.sync_copy(acc.at[pl.ds(s * _RPT, _RPT), :],
                    out.at[pl.ds(c * _NPAD + s * _RPT, _RPT), :])


_mp_kernel = pl.kernel(
    _mp_body,
    out_type=jax.ShapeDtypeStruct((2 * _NPAD, _D), jnp.float32),
    mesh=_sc_mesh,
    scratch_types=[
        pltpu.VMEM((_CPW, _CHUNK), jnp.int32),
        pltpu.VMEM((_CPW, _CHUNK), jnp.int32),
        pltpu.VMEM((_CHUNK, _D), jnp.float32),
        pltpu.VMEM_SHARED((_NPAD, _D), jnp.float32),
        pltpu.SemaphoreType.DMA,
    ],
)

_RB = 1000


def _tc1_body(x, w, dg, out):
    norm = lax.rsqrt(jnp.maximum(dg[...], 1.0))
    out[...] = jnp.dot(x[...], w[...], preferred_element_type=jnp.float32) * norm


_tc1 = pl.pallas_call(
    _tc1_body,
    grid=(_N // _RB,),
    in_specs=[
        pl.BlockSpec((_RB, _D), lambda i: (i, 0)),
        pl.BlockSpec((_D, _D), lambda i: (0, 0)),
        pl.BlockSpec((_RB, 1), lambda i: (i, 0)),
    ],
    out_specs=pl.BlockSpec((_RB, _D), lambda i: (i, 0)),
    out_shape=jax.ShapeDtypeStruct((_N, _D), jnp.float32),
)


def _tc2_body(p0, p1, di, b, w, do, out):
    m = (p0[...] + p1[...]) * lax.rsqrt(jnp.maximum(di[...], 1.0)) + b[...]
    h = jnp.maximum(m, 0.0)
    out[...] = jnp.dot(h, w[...], preferred_element_type=jnp.float32) \
        * lax.rsqrt(jnp.maximum(do[...], 1.0))


_tc2 = pl.pallas_call(
    _tc2_body,
    grid=(_N // _RB,),
    in_specs=[
        pl.BlockSpec((_RB, _D), lambda i: (i, 0)),
        pl.BlockSpec((_RB, _D), lambda i: (i, 0)),
        pl.BlockSpec((_RB, 1), lambda i: (i, 0)),
        pl.BlockSpec((1, _D), lambda i: (0, 0)),
        pl.BlockSpec((_D, _D), lambda i: (0, 0)),
        pl.BlockSpec((_RB, 1), lambda i: (i, 0)),
    ],
    out_specs=pl.BlockSpec((_RB, _D), lambda i: (i, 0)),
    out_shape=jax.ShapeDtypeStruct((_N, _D), jnp.float32),
)


def _tc3_body(p0, p1, di, b, out):
    m = (p0[...] + p1[...]) * lax.rsqrt(jnp.maximum(di[...], 1.0)) + b[...]
    out[...] = jnp.maximum(m, 0.0)


_tc3 = pl.pallas_call(
    _tc3_body,
    grid=(_N // _RB,),
    in_specs=[
        pl.BlockSpec((_RB, _D), lambda i: (i, 0)),
        pl.BlockSpec((_RB, _D), lambda i: (i, 0)),
        pl.BlockSpec((_RB, 1), lambda i: (i, 0)),
        pl.BlockSpec((1, _D), lambda i: (0, 0)),
    ],
    out_specs=pl.BlockSpec((_RB, _D), lambda i: (i, 0)),
    out_shape=jax.ShapeDtypeStruct((_N, _D), jnp.float32),
)


def kernel(features, edge_index, W1, b1, W2, b2):
    src = edge_index[0]
    dst = edge_index[1]
    pad = _N + (jnp.arange(_EPAD - _E, dtype=jnp.int32) % (_NPAD - _N))
    src_m = jnp.concatenate([src, pad]).reshape(_NW * _CPW, _CHUNK)
    dst_m = jnp.concatenate([dst, pad]).reshape(_NW * _CPW, _CHUNK)
    em = jnp.concatenate([src_m, dst_m], axis=0)
    zn = jnp.zeros((_NPAD,), jnp.float32)
    znd = jnp.zeros((_NPAD, _D), jnp.float32)
    rowpad = jnp.zeros((_NPAD - _N, _D), jnp.float32)

    degs = _deg_kernel(em, zn)
    deg_out = degs[:_N].reshape(_N, 1)
    deg_in = degs[_NPAD:_NPAD + _N].reshape(_N, 1)

    hs1 = _tc1(features, W1, deg_out)
    parts1 = _mp_kernel(jnp.concatenate([hs1, rowpad], axis=0), src_m, dst_m, znd)

    hs2 = _tc2(parts1[:_N], parts1[_NPAD:_NPAD + _N], deg_in,
               b1.reshape(1, _D), W2, deg_out)
    parts2 = _mp_kernel(jnp.concatenate([hs2, rowpad], axis=0), src_m, dst_m, znd)

    return _tc3(parts2[:_N], parts2[_NPAD:_NPAD + _N], deg_in, b2.reshape(1, _D))


# speedup vs baseline: 10.6885x; 1.3835x over previous
import jax
import jax.numpy as jnp
from jax import lax
from jax.experimental import pallas as pl
from jax.experimental.pallas import tpu as pltpu
from jax.experimental.pallas import tpu_sc as plsc

_N = 10000
_D = 128
_E = 320000
_NC, _NS = 2, 16
_NW = _NC * _NS
_CHUNK = 128
_CPW = 80
_NPAD = _CPW * _CHUNK
_EPAD = _NW * _CPW * _CHUNK
_RPT = _NPAD // _NS
_DEG_ROWS = 2 * _EPAD // _CHUNK
_DEG_RPT = _DEG_ROWS // _NW

_sc_mesh = plsc.VectorSubcoreMesh(core_axis_name="c", subcore_axis_name="s")


def _deg_body(em, zn, out, idx, ones_v, acc, dsem):
    c = lax.axis_index("c")
    s = lax.axis_index("s")
    base = (c * _NS + s) * _DEG_RPT
    pltpu.sync_copy(zn.at[pl.ds(s * _RPT, _RPT)], acc.at[pl.ds(s * _RPT, _RPT)])
    pltpu.sync_copy(em.at[pl.ds(base, _DEG_RPT), :], idx)
    for i in range(_CHUNK // 16):
        ones_v[pl.ds(i * 16, 16)] = jnp.full((16,), 1.0, jnp.float32)
    plsc.subcore_barrier()

    def group(g, carry):
        for b in range(8):
            pltpu.async_copy(ones_v, acc.at[idx.at[g * 8 + b]], dsem, add=True)
        for b in range(8):
            pltpu.make_async_copy(ones_v, acc.at[idx.at[g * 8 + b]],
                                  dsem).wait()
        return carry

    lax.fori_loop(0, _DEG_RPT // 8, group, 0)
    plsc.subcore_barrier()
    pltpu.sync_copy(acc.at[pl.ds(s * _RPT, _RPT)],
                    out.at[pl.ds(c * _NPAD + s * _RPT, _RPT)])


_deg_kernel = pl.kernel(
    _deg_body,
    out_type=jax.ShapeDtypeStruct((2 * _NPAD,), jnp.float32),
    mesh=_sc_mesh,
    scratch_types=[
        pltpu.VMEM((_DEG_RPT, _CHUNK), jnp.int32),
        pltpu.VMEM((_CHUNK,), jnp.float32),
        pltpu.VMEM_SHARED((_NPAD,), jnp.float32),
        pltpu.SemaphoreType.DMA,
    ],
)


_NBUF = 4
_DH = _D // 2
_CPT = _EPAD // _CHUNK // _NS


def _mp_body(hs, srcb, dstm, zz, out, idx_s, idx_d, rows, acc, gsem, ssem):
    c = lax.axis_index("c")
    s = lax.axis_index("s")
    base = s * _CPT
    pltpu.sync_copy(zz.at[pl.ds(s * _RPT, _RPT), :],
                    acc.at[pl.ds(s * _RPT, _RPT), :])
    pltpu.sync_copy(srcb.at[pl.ds(c * _NS * _CPT + base, _CPT), :], idx_s)
    pltpu.sync_copy(dstm.at[pl.ds(base, _CPT), :], idx_d)
    plsc.subcore_barrier()

    def g_issue(j, b):
        pltpu.async_copy(hs.at[idx_s.at[j]], rows.at[b], gsem.at[b])

    def g_wait(j, b):
        pltpu.make_async_copy(hs.at[idx_s.at[j]], rows.at[b], gsem.at[b]).wait()

    def s_issue(j, b):
        pltpu.async_copy(rows.at[b], acc.at[idx_d.at[j]], ssem.at[b], add=True)

    def s_wait(j, b):
        pltpu.make_async_copy(rows.at[b], acc.at[idx_d.at[j]],
                              ssem.at[b]).wait()

    g_issue(0, 0)
    g_issue(1, 1)
    g_issue(2, 2)
    g_wait(0, 0)
    s_issue(0, 0)
    g_issue(3, 3)
    g_wait(1, 1)
    s_issue(1, 1)

    def group(g, carry):
        j0 = g * _NBUF
        for b in range(_NBUF):
            j = j0 + b
            s_wait(j - 4, b)
            g_issue(j, b)
            b2 = (b + 2) % _NBUF
            g_wait(j - 2, b2)
            s_issue(j - 2, b2)
        return carry

    lax.fori_loop(1, _CPT // _NBUF, group, 0)

    g_wait(_CPT - 2, (_CPT - 2) % _NBUF)
    s_issue(_CPT - 2, (_CPT - 2) % _NBUF)
    g_wait(_CPT - 1, (_CPT - 1) % _NBUF)
    s_issue(_CPT - 1, (_CPT - 1) % _NBUF)
    for j in range(_CPT - 4, _CPT):
        s_wait(j, j % _NBUF)
    plsc.subcore_barrier()
    pltpu.sync_copy(acc.at[pl.ds(s * _RPT, _RPT), :],
                    out.at[pl.ds(c * _NPAD + s * _RPT, _RPT), :])


_mp_kernel = pl.kernel(
    _mp_body,
    out_type=jax.ShapeDtypeStruct((2 * _NPAD, _DH), jnp.float32),
    mesh=_sc_mesh,
    scratch_types=[
        pltpu.VMEM((_CPT, _CHUNK), jnp.int32),
        pltpu.VMEM((_CPT, _CHUNK), jnp.int32),
        pltpu.VMEM((_NBUF, _CHUNK, _DH), jnp.float32),
        pltpu.VMEM_SHARED((_NPAD, _DH), jnp.float32),
        pltpu.SemaphoreType.DMA((_NBUF,)),
        pltpu.SemaphoreType.DMA((_NBUF,)),
    ],
    compiler_params=pltpu.CompilerParams(use_tc_tiling_on_sc=False),
)

_RB = 1000


def _tc1_body(x, w, dg, out):
    norm = lax.rsqrt(jnp.maximum(dg[...], 1.0))
    r = jnp.dot(x[...], w[...], preferred_element_type=jnp.float32) * norm
    out[0, :, :] = r[:, :_DH]
    out[1, :, :] = r[:, _DH:]


_tc1 = pl.pallas_call(
    _tc1_body,
    grid=(_N // _RB,),
    in_specs=[
        pl.BlockSpec((_RB, _D), lambda i: (i, 0)),
        pl.BlockSpec((_D, _D), lambda i: (0, 0)),
        pl.BlockSpec((_RB, 1), lambda i: (i, 0)),
    ],
    out_specs=pl.BlockSpec((2, _RB, _DH), lambda i: (0, i, 0)),
    out_shape=jax.ShapeDtypeStruct((2, _N, _DH), jnp.float32),
)


def _tc2_body(p0, p1, di, b, w, do, out):
    p = jnp.concatenate([p0[...], p1[...]], axis=1)
    m = p * lax.rsqrt(jnp.maximum(di[...], 1.0)) + b[...]
    h = jnp.maximum(m, 0.0)
    r = jnp.dot(h, w[...], preferred_element_type=jnp.float32) \
        * lax.rsqrt(jnp.maximum(do[...], 1.0))
    out[0, :, :] = r[:, :_DH]
    out[1, :, :] = r[:, _DH:]


_tc2 = pl.pallas_call(
    _tc2_body,
    grid=(_N // _RB,),
    in_specs=[
        pl.BlockSpec((_RB, _DH), lambda i: (i, 0)),
        pl.BlockSpec((_RB, _DH), lambda i: (i, 0)),
        pl.BlockSpec((_RB, 1), lambda i: (i, 0)),
        pl.BlockSpec((1, _D), lambda i: (0, 0)),
        pl.BlockSpec((_D, _D), lambda i: (0, 0)),
        pl.BlockSpec((_RB, 1), lambda i: (i, 0)),
    ],
    out_specs=pl.BlockSpec((2, _RB, _DH), lambda i: (0, i, 0)),
    out_shape=jax.ShapeDtypeStruct((2, _N, _DH), jnp.float32),
)


def _tc3_body(p0, p1, di, b, out):
    p = jnp.concatenate([p0[...], p1[...]], axis=1)
    m = p * lax.rsqrt(jnp.maximum(di[...], 1.0)) + b[...]
    out[...] = jnp.maximum(m, 0.0)


_tc3 = pl.pallas_call(
    _tc3_body,
    grid=(_N // _RB,),
    in_specs=[
        pl.BlockSpec((_RB, _DH), lambda i: (i, 0)),
        pl.BlockSpec((_RB, _DH), lambda i: (i, 0)),
        pl.BlockSpec((_RB, 1), lambda i: (i, 0)),
        pl.BlockSpec((1, _D), lambda i: (0, 0)),
    ],
    out_specs=pl.BlockSpec((_RB, _D), lambda i: (i, 0)),
    out_shape=jax.ShapeDtypeStruct((_N, _D), jnp.float32),
)


def kernel(features, edge_index, W1, b1, W2, b2):
    src = edge_index[0]
    dst = edge_index[1]
    spread = jnp.arange(_EPAD - _E, dtype=jnp.int32) % (_NPAD - _N)
    src_mp = jnp.concatenate([src, spread]).reshape(_NW * _CPW, _CHUNK)
    src_dg = jnp.concatenate([src, _N + spread]).reshape(_NW * _CPW, _CHUNK)
    dst_m = jnp.concatenate([dst, _N + spread]).reshape(_NW * _CPW, _CHUNK)
    srcb = jnp.concatenate([src_mp, src_mp + _N], axis=0)
    em = jnp.concatenate([src_dg, dst_m], axis=0)
    zn = jnp.zeros((_NPAD,), jnp.float32)
    znd = jnp.zeros((_NPAD, _DH), jnp.float32)

    degs = _deg_kernel(em, zn)
    deg_out = degs[:_N].reshape(_N, 1)
    deg_in = degs[_NPAD:_NPAD + _N].reshape(_N, 1)

    hs1 = _tc1(features, W1, deg_out).reshape(2 * _N, _DH)
    parts1 = _mp_kernel(hs1, srcb, dst_m, znd)

    hs2 = _tc2(parts1[:_N], parts1[_NPAD:_NPAD + _N], deg_in,
               b1.reshape(1, _D), W2, deg_out).reshape(2 * _N, _DH)
    parts2 = _mp_kernel(hs2, srcb, dst_m, znd)

    return _tc3(parts2[:_N], parts2[_NPAD:_NPAD + _N], deg_in, b2.reshape(1, _D))


# speedup vs baseline: 12.4225x; 1.1622x over previous
import jax
import jax.numpy as jnp
from jax import lax
from jax.experimental import pallas as pl
from jax.experimental.pallas import tpu as pltpu
from jax.experimental.pallas import tpu_sc as plsc

_N = 10000
_D = 128
_E = 320000
_NC, _NS = 2, 16
_NW = _NC * _NS
_CHUNK = 128
_CPW = 80
_NPAD = _CPW * _CHUNK
_EPAD = _NW * _CPW * _CHUNK
_RPT = _NPAD // _NS
_DEG_ROWS = 2 * _EPAD // _CHUNK
_DEG_RPT = _DEG_ROWS // _NW

_sc_mesh = plsc.VectorSubcoreMesh(core_axis_name="c", subcore_axis_name="s")


def _deg_body(em, zn, out, idx, ones_v, acc, dsem):
    c = lax.axis_index("c")
    s = lax.axis_index("s")
    base = (c * _NS + s) * _DEG_RPT
    pltpu.sync_copy(zn.at[pl.ds(s * _RPT, _RPT)], acc.at[pl.ds(s * _RPT, _RPT)])
    pltpu.sync_copy(em.at[pl.ds(base, _DEG_RPT), :], idx)
    for i in range(_CHUNK // 16):
        ones_v[pl.ds(i * 16, 16)] = jnp.full((16,), 1.0, jnp.float32)
    plsc.subcore_barrier()

    def group(g, carry):
        for b in range(8):
            pltpu.async_copy(ones_v, acc.at[idx.at[g * 8 + b]], dsem, add=True)
        for b in range(8):
            pltpu.make_async_copy(ones_v, acc.at[idx.at[g * 8 + b]],
                                  dsem).wait()
        return carry

    lax.fori_loop(0, _DEG_RPT // 8, group, 0)
    plsc.subcore_barrier()
    pltpu.sync_copy(acc.at[pl.ds(s * _RPT, _RPT)],
                    out.at[pl.ds(c * _NPAD + s * _RPT, _RPT)])


_deg_kernel = pl.kernel(
    _deg_body,
    out_type=jax.ShapeDtypeStruct((2 * _NPAD,), jnp.float32),
    mesh=_sc_mesh,
    scratch_types=[
        pltpu.VMEM((_DEG_RPT, _CHUNK), jnp.int32),
        pltpu.VMEM((_CHUNK,), jnp.float32),
        pltpu.VMEM_SHARED((_NPAD,), jnp.float32),
        pltpu.SemaphoreType.DMA,
    ],
)


_NBUF = 4
_DH = _D // 2
_CPT = _EPAD // _CHUNK // _NS


def _mp_body(hs, srcb, dstm, zz, out, idx_s, idx_d, rows, acc, gsem, ssem):
    c = lax.axis_index("c")
    s = lax.axis_index("s")
    base = s * _CPT
    pltpu.sync_copy(zz.at[pl.ds(s * _RPT, _RPT), :],
                    acc.at[pl.ds(s * _RPT, _RPT), :])
    pltpu.sync_copy(srcb.at[pl.ds(c * _NS * _CPT + base, _CPT), :], idx_s)
    pltpu.sync_copy(dstm.at[pl.ds(base, _CPT), :], idx_d)
    plsc.subcore_barrier()

    def g_issue(j, b):
        pltpu.async_copy(hs.at[idx_s.at[j]], rows.at[b], gsem.at[b])

    def g_wait(j, b):
        pltpu.make_async_copy(hs.at[idx_s.at[j]], rows.at[b], gsem.at[b]).wait()

    def s_issue(j, b):
        pltpu.async_copy(rows.at[b], acc.at[idx_d.at[j]], ssem.at[b], add=True)

    def s_wait(j, b):
        pltpu.make_async_copy(rows.at[b], acc.at[idx_d.at[j]],
                              ssem.at[b]).wait()

    g_issue(0, 0)
    g_issue(1, 1)
    g_issue(2, 2)
    g_wait(0, 0)
    s_issue(0, 0)
    g_issue(3, 3)
    g_wait(1, 1)
    s_issue(1, 1)

    def group(g, carry):
        j0 = g * _NBUF
        for b in range(_NBUF):
            j = j0 + b
            s_wait(j - 4, b)
            g_issue(j, b)
            b2 = (b + 2) % _NBUF
            g_wait(j - 2, b2)
            s_issue(j - 2, b2)
        return carry

    lax.fori_loop(1, _CPT // _NBUF, group, 0)

    g_wait(_CPT - 2, (_CPT - 2) % _NBUF)
    s_issue(_CPT - 2, (_CPT - 2) % _NBUF)
    g_wait(_CPT - 1, (_CPT - 1) % _NBUF)
    s_issue(_CPT - 1, (_CPT - 1) % _NBUF)
    for j in range(_CPT - 4, _CPT):
        s_wait(j, j % _NBUF)
    plsc.subcore_barrier()
    pltpu.sync_copy(acc.at[pl.ds(s * _RPT, _RPT), :],
                    out.at[pl.ds(s * _RPT, _RPT), pl.ds(c * _DH, _DH)])


_mp_kernel = pl.kernel(
    _mp_body,
    out_type=jax.ShapeDtypeStruct((_NPAD, _D), jnp.float32),
    mesh=_sc_mesh,
    scratch_types=[
        pltpu.VMEM((_CPT, _CHUNK), jnp.int32),
        pltpu.VMEM((_CPT, _CHUNK), jnp.int32),
        pltpu.VMEM((_NBUF, _CHUNK, _DH), jnp.float32),
        pltpu.VMEM_SHARED((_NPAD, _DH), jnp.float32),
        pltpu.SemaphoreType.DMA((_NBUF,)),
        pltpu.SemaphoreType.DMA((_NBUF,)),
    ],
    compiler_params=pltpu.CompilerParams(use_tc_tiling_on_sc=False),
)

_RB = 1000


def _tc1_body(x, w, dg, out):
    norm = lax.rsqrt(jnp.maximum(dg[...], 1.0))
    out[...] = jnp.dot(x[...], w[...], preferred_element_type=jnp.float32) * norm


_tc1 = pl.pallas_call(
    _tc1_body,
    grid=(_N // _RB,),
    in_specs=[
        pl.BlockSpec((_RB, _D), lambda i: (i, 0)),
        pl.BlockSpec((_D, _D), lambda i: (0, 0)),
        pl.BlockSpec((_RB, 1), lambda i: (i, 0)),
    ],
    out_specs=pl.BlockSpec((_RB, _D), lambda i: (i, 0)),
    out_shape=jax.ShapeDtypeStruct((_N, _D), jnp.float32),
)


def _tc2_body(p, di, b, w, do, out):
    m = p[...] * lax.rsqrt(jnp.maximum(di[...], 1.0)) + b[...]
    h = jnp.maximum(m, 0.0)
    out[...] = jnp.dot(h, w[...], preferred_element_type=jnp.float32) \
        * lax.rsqrt(jnp.maximum(do[...], 1.0))


_tc2 = pl.pallas_call(
    _tc2_body,
    grid=(_N // _RB,),
    in_specs=[
        pl.BlockSpec((_RB, _D), lambda i: (i, 0)),
        pl.BlockSpec((_RB, 1), lambda i: (i, 0)),
        pl.BlockSpec((1, _D), lambda i: (0, 0)),
        pl.BlockSpec((_D, _D), lambda i: (0, 0)),
        pl.BlockSpec((_RB, 1), lambda i: (i, 0)),
    ],
    out_specs=pl.BlockSpec((_RB, _D), lambda i: (i, 0)),
    out_shape=jax.ShapeDtypeStruct((_N, _D), jnp.float32),
)


def _tc3_body(p, di, b, out):
    m = p[...] * lax.rsqrt(jnp.maximum(di[...], 1.0)) + b[...]
    out[...] = jnp.maximum(m, 0.0)


_tc3 = pl.pallas_call(
    _tc3_body,
    grid=(_N // _RB,),
    in_specs=[
        pl.BlockSpec((_RB, _D), lambda i: (i, 0)),
        pl.BlockSpec((_RB, 1), lambda i: (i, 0)),
        pl.BlockSpec((1, _D), lambda i: (0, 0)),
    ],
    out_specs=pl.BlockSpec((_RB, _D), lambda i: (i, 0)),
    out_shape=jax.ShapeDtypeStruct((_N, _D), jnp.float32),
)


def kernel(features, edge_index, W1, b1, W2, b2):
    src = edge_index[0]
    dst = edge_index[1]
    spread = jnp.arange(_EPAD - _E, dtype=jnp.int32) % (_NPAD - _N)
    src_mp = jnp.concatenate([src, spread]).reshape(_NW * _CPW, _CHUNK)
    src_dg = jnp.concatenate([src, _N + spread]).reshape(_NW * _CPW, _CHUNK)
    dst_m = jnp.concatenate([dst, _N + spread]).reshape(_NW * _CPW, _CHUNK)
    srcb = jnp.concatenate([2 * src_mp, 2 * src_mp + 1], axis=0)
    em = jnp.concatenate([src_dg, dst_m], axis=0)
    zn = jnp.zeros((_NPAD,), jnp.float32)
    znd = jnp.zeros((_NPAD, _DH), jnp.float32)

    degs = _deg_kernel(em, zn)
    deg_out = degs[:_N].reshape(_N, 1)
    deg_in = degs[_NPAD:_NPAD + _N].reshape(_N, 1)

    hs1 = _tc1(features, W1, deg_out).reshape(2 * _N, _DH)
    parts1 = _mp_kernel(hs1, srcb, dst_m, znd)

    hs2 = _tc2(parts1[:_N], deg_in, b1.reshape(1, _D), W2,
               deg_out).reshape(2 * _N, _DH)
    parts2 = _mp_kernel(hs2, srcb, dst_m, znd)

    return _tc3(parts2[:_N], deg_in, b2.reshape(1, _D))


# speedup vs baseline: 12.8074x; 1.0310x over previous
import jax
import jax.numpy as jnp
from jax import lax
from jax.experimental import pallas as pl
from jax.experimental.pallas import tpu as pltpu
from jax.experimental.pallas import tpu_sc as plsc

_N = 10000
_D = 128
_E = 320000
_NC, _NS = 2, 16
_NW = _NC * _NS
_CHUNK = 128
_CPW = 80
_NPAD = _CPW * _CHUNK
_EPAD = _NW * _CPW * _CHUNK
_RPT = _NPAD // _NS
_DEG_ROWS = 2 * _EPAD // _CHUNK
_DEG_RPT = _DEG_ROWS // _NW

_sc_mesh = plsc.VectorSubcoreMesh(core_axis_name="c", subcore_axis_name="s")


def _deg_body(em, zn, out, idx, ones_v, acc, dsem):
    c = lax.axis_index("c")
    s = lax.axis_index("s")
    base = (c * _NS + s) * _DEG_RPT
    pltpu.sync_copy(zn.at[pl.ds(s * _RPT, _RPT)], acc.at[pl.ds(s * _RPT, _RPT)])
    pltpu.sync_copy(em.at[pl.ds(base, _DEG_RPT), :], idx)
    for i in range(_CHUNK // 16):
        ones_v[pl.ds(i * 16, 16)] = jnp.full((16,), 1.0, jnp.float32)
    plsc.subcore_barrier()

    def group(g, carry):
        for b in range(8):
            pltpu.async_copy(ones_v, acc.at[idx.at[g * 8 + b]], dsem, add=True)
        for b in range(8):
            pltpu.make_async_copy(ones_v, acc.at[idx.at[g * 8 + b]],
                                  dsem).wait()
        return carry

    lax.fori_loop(0, _DEG_RPT // 8, group, 0)
    plsc.subcore_barrier()
    pltpu.sync_copy(acc.at[pl.ds(s * _RPT, _RPT)],
                    out.at[pl.ds(c * _NPAD + s * _RPT, _RPT)])


_deg_kernel = pl.kernel(
    _deg_body,
    out_type=jax.ShapeDtypeStruct((2 * _NPAD,), jnp.float32),
    mesh=_sc_mesh,
    scratch_types=[
        pltpu.VMEM((_DEG_RPT, _CHUNK), jnp.int32),
        pltpu.VMEM((_CHUNK,), jnp.float32),
        pltpu.VMEM_SHARED((_NPAD,), jnp.float32),
        pltpu.SemaphoreType.DMA,
    ],
)


_NBUF = 5
_DH = _D // 2
_CPT = _EPAD // _CHUNK // _NS


def _mp_body(hs, srcb, dstm, zz, out, idx_s, idx_d, rows, acc, gsem, ssem):
    c = lax.axis_index("c")
    s = lax.axis_index("s")
    base = s * _CPT
    pltpu.sync_copy(zz.at[pl.ds(s * _RPT, _RPT), :],
                    acc.at[pl.ds(s * _RPT, _RPT), :])
    pltpu.sync_copy(srcb.at[pl.ds(c * _NS * _CPT + base, _CPT), :], idx_s)
    pltpu.sync_copy(dstm.at[pl.ds(base, _CPT), :], idx_d)
    plsc.subcore_barrier()

    def g_issue(j, b):
        pltpu.async_copy(hs.at[idx_s.at[j]], rows.at[b], gsem.at[b])

    def g_wait(j, b):
        pltpu.make_async_copy(hs.at[idx_s.at[j]], rows.at[b], gsem.at[b]).wait()

    def s_issue(j, b):
        pltpu.async_copy(rows.at[b], acc.at[idx_d.at[j]], ssem.at[b], add=True)

    def s_wait(j, b):
        pltpu.make_async_copy(rows.at[b], acc.at[idx_d.at[j]],
                              ssem.at[b]).wait()

    g_issue(0, 0)
    g_issue(1, 1)
    g_issue(2, 2)
    g_wait(0, 0)
    s_issue(0, 0)
    g_issue(3, 3)
    g_wait(1, 1)
    s_issue(1, 1)
    g_issue(4, 4)
    g_wait(2, 2)
    s_issue(2, 2)

    def group(g, carry):
        j0 = g * _NBUF
        for b in range(_NBUF):
            j = j0 + b
            s_wait(j - _NBUF, b)
            g_issue(j, b)
            b2 = (b + _NBUF - 2) % _NBUF
            g_wait(j - 2, b2)
            s_issue(j - 2, b2)
        return carry

    lax.fori_loop(1, _CPT // _NBUF, group, 0)

    g_wait(_CPT - 2, (_CPT - 2) % _NBUF)
    s_issue(_CPT - 2, (_CPT - 2) % _NBUF)
    g_wait(_CPT - 1, (_CPT - 1) % _NBUF)
    s_issue(_CPT - 1, (_CPT - 1) % _NBUF)
    for j in range(_CPT - _NBUF, _CPT):
        s_wait(j, j % _NBUF)
    plsc.subcore_barrier()
    pltpu.sync_copy(acc.at[pl.ds(s * _RPT, _RPT), :],
                    out.at[pl.ds(s * _RPT, _RPT), pl.ds(c * _DH, _DH)])


_mp_kernel = pl.kernel(
    _mp_body,
    out_type=jax.ShapeDtypeStruct((_NPAD, _D), jnp.float32),
    mesh=_sc_mesh,
    scratch_types=[
        pltpu.VMEM((_CPT, _CHUNK), jnp.int32),
        pltpu.VMEM((_CPT, _CHUNK), jnp.int32),
        pltpu.VMEM((_NBUF, _CHUNK, _DH), jnp.float32),
        pltpu.VMEM_SHARED((_NPAD, _DH), jnp.float32),
        pltpu.SemaphoreType.DMA((_NBUF,)),
        pltpu.SemaphoreType.DMA((_NBUF,)),
    ],
    compiler_params=pltpu.CompilerParams(use_tc_tiling_on_sc=False),
)

_RB = 1000


def _tc1_body(x, w, dg, out):
    norm = lax.rsqrt(jnp.maximum(dg[...], 1.0))
    out[...] = jnp.dot(x[...], w[...], preferred_element_type=jnp.float32) * norm


_tc1 = pl.pallas_call(
    _tc1_body,
    grid=(_N // _RB,),
    in_specs=[
        pl.BlockSpec((_RB, _D), lambda i: (i, 0)),
        pl.BlockSpec((_D, _D), lambda i: (0, 0)),
        pl.BlockSpec((_RB, 1), lambda i: (i, 0)),
    ],
    out_specs=pl.BlockSpec((_RB, _D), lambda i: (i, 0)),
    out_shape=jax.ShapeDtypeStruct((_N, _D), jnp.float32),
)


def _tc2_body(p, di, b, w, do, out):
    m = p[...] * lax.rsqrt(jnp.maximum(di[...], 1.0)) + b[...]
    h = jnp.maximum(m, 0.0)
    out[...] = jnp.dot(h, w[...], preferred_element_type=jnp.float32) \
        * lax.rsqrt(jnp.maximum(do[...], 1.0))


_tc2 = pl.pallas_call(
    _tc2_body,
    grid=(_N // _RB,),
    in_specs=[
        pl.BlockSpec((_RB, _D), lambda i: (i, 0)),
        pl.BlockSpec((_RB, 1), lambda i: (i, 0)),
        pl.BlockSpec((1, _D), lambda i: (0, 0)),
        pl.BlockSpec((_D, _D), lambda i: (0, 0)),
        pl.BlockSpec((_RB, 1), lambda i: (i, 0)),
    ],
    out_specs=pl.BlockSpec((_RB, _D), lambda i: (i, 0)),
    out_shape=jax.ShapeDtypeStruct((_N, _D), jnp.float32),
)


def _tc3_body(p, di, b, out):
    m = p[...] * lax.rsqrt(jnp.maximum(di[...], 1.0)) + b[...]
    out[...] = jnp.maximum(m, 0.0)


_tc3 = pl.pallas_call(
    _tc3_body,
    grid=(_N // _RB,),
    in_specs=[
        pl.BlockSpec((_RB, _D), lambda i: (i, 0)),
        pl.BlockSpec((_RB, 1), lambda i: (i, 0)),
        pl.BlockSpec((1, _D), lambda i: (0, 0)),
    ],
    out_specs=pl.BlockSpec((_RB, _D), lambda i: (i, 0)),
    out_shape=jax.ShapeDtypeStruct((_N, _D), jnp.float32),
)


def kernel(features, edge_index, W1, b1, W2, b2):
    src = edge_index[0]
    dst = edge_index[1]
    spread = jnp.arange(_EPAD - _E, dtype=jnp.int32) % (_NPAD - _N)
    src_mp = jnp.concatenate([src, spread]).reshape(_NW * _CPW, _CHUNK)
    src_dg = jnp.concatenate([src, _N + spread]).reshape(_NW * _CPW, _CHUNK)
    dst_m = jnp.concatenate([dst, _N + spread]).reshape(_NW * _CPW, _CHUNK)
    srcb = jnp.concatenate([2 * src_mp, 2 * src_mp + 1], axis=0)
    em = jnp.concatenate([src_dg, dst_m], axis=0)
    zn = jnp.zeros((_NPAD,), jnp.float32)
    znd = jnp.zeros((_NPAD, _DH), jnp.float32)

    degs = _deg_kernel(em, zn)
    deg_out = degs[:_N].reshape(_N, 1)
    deg_in = degs[_NPAD:_NPAD + _N].reshape(_N, 1)

    hs1 = _tc1(features, W1, deg_out).reshape(2 * _N, _DH)
    parts1 = _mp_kernel(hs1, srcb, dst_m, znd)

    hs2 = _tc2(parts1[:_N], deg_in, b1.reshape(1, _D), W2,
               deg_out).reshape(2 * _N, _DH)
    parts2 = _mp_kernel(hs2, srcb, dst_m, znd)

    return _tc3(parts2[:_N], deg_in, b2.reshape(1, _D))


# speedup vs baseline: 15.6663x; 1.2232x over previous
import jax
import jax.numpy as jnp
from jax import lax
from jax.experimental import pallas as pl
from jax.experimental.pallas import tpu as pltpu
from jax.experimental.pallas import tpu_sc as plsc

_N = 10000
_D = 128
_E = 320000
_NC, _NS = 2, 16
_NW = _NC * _NS
_CHUNK = 128
_CPW = 80
_NPAD = _CPW * _CHUNK
_EPAD = _NW * _CPW * _CHUNK
_RPT = _NPAD // _NS
_DEG_ROWS = 2 * _EPAD // _CHUNK
_DEG_RPT = _DEG_ROWS // _NW

_sc_mesh = plsc.VectorSubcoreMesh(core_axis_name="c", subcore_axis_name="s")


def _deg_body(em, zn, out, idx, ones_v, acc, dsem):
    c = lax.axis_index("c")
    s = lax.axis_index("s")
    base = (c * _NS + s) * _DEG_RPT
    pltpu.sync_copy(zn.at[pl.ds(s * _RPT, _RPT)], acc.at[pl.ds(s * _RPT, _RPT)])
    pltpu.sync_copy(em.at[pl.ds(base, _DEG_RPT), :], idx)
    for i in range(_CHUNK // 16):
        ones_v[pl.ds(i * 16, 16)] = jnp.full((16,), 1.0, jnp.float32)
    plsc.subcore_barrier()

    def group(g, carry):
        for b in range(8):
            pltpu.async_copy(ones_v, acc.at[idx.at[g * 8 + b]], dsem, add=True)
        for b in range(8):
            pltpu.make_async_copy(ones_v, acc.at[idx.at[g * 8 + b]],
                                  dsem).wait()
        return carry

    lax.fori_loop(0, _DEG_RPT // 8, group, 0)
    plsc.subcore_barrier()
    pltpu.sync_copy(acc.at[pl.ds(s * _RPT, _RPT)],
                    out.at[pl.ds(c * _NPAD + s * _RPT, _RPT)])


_deg_kernel = pl.kernel(
    _deg_body,
    out_type=jax.ShapeDtypeStruct((2 * _NPAD,), jnp.float32),
    mesh=_sc_mesh,
    scratch_types=[
        pltpu.VMEM((_DEG_RPT, _CHUNK), jnp.int32),
        pltpu.VMEM((_CHUNK,), jnp.float32),
        pltpu.VMEM_SHARED((_NPAD,), jnp.float32),
        pltpu.SemaphoreType.DMA,
    ],
)


_NBUF = 8
_LG = 3
_DH = _D // 2
_CPT = _EPAD // _CHUNK // _NS


def _mp_body(hs, srcb, dstm, zz, out, idx_s, idx_d, rows, acc, gsem, ssem):
    c = lax.axis_index("c")
    s = lax.axis_index("s")
    base = s * _CPT
    pltpu.sync_copy(zz.at[pl.ds(s * _RPT, _RPT), :],
                    acc.at[pl.ds(s * _RPT, _RPT), :])
    pltpu.sync_copy(srcb.at[pl.ds(c * _NS * _CPT + base, _CPT), :], idx_s)
    pltpu.sync_copy(dstm.at[pl.ds(base, _CPT), :], idx_d)
    plsc.subcore_barrier()

    def g_issue(j, b):
        pltpu.async_copy(hs.at[idx_s.at[j]], rows.at[b], gsem.at[b])

    def g_wait(j, b):
        pltpu.make_async_copy(hs.at[idx_s.at[j]], rows.at[b], gsem.at[b]).wait()

    def s_issue(j, b):
        pltpu.async_copy(rows.at[b], acc.at[idx_d.at[j]], ssem.at[b], add=True)

    def s_wait(j, b):
        pltpu.make_async_copy(rows.at[b], acc.at[idx_d.at[j]],
                              ssem.at[b]).wait()

    for b in range(_NBUF):
        g_issue(b, b)
    for k in range(_NBUF - _LG):
        g_wait(k, k)
        s_issue(k, k)

    def group(g, carry):
        j0 = g * _NBUF
        for b in range(_NBUF):
            j = j0 + b
            s_wait(j - _NBUF, b)
            g_issue(j, b)
            b2 = (b + _NBUF - _LG) % _NBUF
            g_wait(j - _LG, b2)
            s_issue(j - _LG, b2)
        return carry

    lax.fori_loop(1, _CPT // _NBUF, group, 0)

    for j in range(_CPT - _LG, _CPT):
        g_wait(j, j % _NBUF)
        s_issue(j, j % _NBUF)
    for j in range(_CPT - _NBUF, _CPT):
        s_wait(j, j % _NBUF)
    plsc.subcore_barrier()
    pltpu.sync_copy(acc.at[pl.ds(s * _RPT, _RPT), :],
                    out.at[pl.ds(s * _RPT, _RPT), pl.ds(c * _DH, _DH)])


_mp_kernel = pl.kernel(
    _mp_body,
    out_type=jax.ShapeDtypeStruct((_NPAD, _D), jnp.bfloat16),
    mesh=_sc_mesh,
    scratch_types=[
        pltpu.VMEM((_CPT, _CHUNK), jnp.int32),
        pltpu.VMEM((_CPT, _CHUNK), jnp.int32),
        pltpu.VMEM((_NBUF, _CHUNK, _DH), jnp.bfloat16),
        pltpu.VMEM_SHARED((_NPAD, _DH), jnp.bfloat16),
        pltpu.SemaphoreType.DMA((_NBUF,)),
        pltpu.SemaphoreType.DMA((_NBUF,)),
    ],
    compiler_params=pltpu.CompilerParams(use_tc_tiling_on_sc=False),
)

_RB = 2000


def _tc1_body(x, w, dg, out):
    norm = lax.rsqrt(jnp.maximum(dg[...], 1.0))
    r = jnp.dot(x[...], w[...], preferred_element_type=jnp.float32) * norm
    out[...] = r.astype(jnp.bfloat16)


_tc1 = pl.pallas_call(
    _tc1_body,
    grid=(_N // _RB,),
    in_specs=[
        pl.BlockSpec((_RB, _D), lambda i: (i, 0)),
        pl.BlockSpec((_D, _D), lambda i: (0, 0)),
        pl.BlockSpec((_RB, 1), lambda i: (i, 0)),
    ],
    out_specs=pl.BlockSpec((_RB, _D), lambda i: (i, 0)),
    out_shape=jax.ShapeDtypeStruct((_N, _D), jnp.bfloat16),
)


def _tc2_body(p, di, b, w, do, out):
    m = p[...].astype(jnp.float32) * lax.rsqrt(jnp.maximum(di[...], 1.0)) + b[...]
    h = jnp.maximum(m, 0.0)
    r = jnp.dot(h, w[...], preferred_element_type=jnp.float32) \
        * lax.rsqrt(jnp.maximum(do[...], 1.0))
    out[...] = r.astype(jnp.bfloat16)


_tc2 = pl.pallas_call(
    _tc2_body,
    grid=(_N // _RB,),
    in_specs=[
        pl.BlockSpec((_RB, _D), lambda i: (i, 0)),
        pl.BlockSpec((_RB, 1), lambda i: (i, 0)),
        pl.BlockSpec((1, _D), lambda i: (0, 0)),
        pl.BlockSpec((_D, _D), lambda i: (0, 0)),
        pl.BlockSpec((_RB, 1), lambda i: (i, 0)),
    ],
    out_specs=pl.BlockSpec((_RB, _D), lambda i: (i, 0)),
    out_shape=jax.ShapeDtypeStruct((_N, _D), jnp.bfloat16),
)


def _tc3_body(p, di, b, out):
    m = p[...].astype(jnp.float32) * lax.rsqrt(jnp.maximum(di[...], 1.0)) + b[...]
    out[...] = jnp.maximum(m, 0.0)


_tc3 = pl.pallas_call(
    _tc3_body,
    grid=(_N // _RB,),
    in_specs=[
        pl.BlockSpec((_RB, _D), lambda i: (i, 0)),
        pl.BlockSpec((_RB, 1), lambda i: (i, 0)),
        pl.BlockSpec((1, _D), lambda i: (0, 0)),
    ],
    out_specs=pl.BlockSpec((_RB, _D), lambda i: (i, 0)),
    out_shape=jax.ShapeDtypeStruct((_N, _D), jnp.float32),
)


def kernel(features, edge_index, W1, b1, W2, b2):
    src = edge_index[0]
    dst = edge_index[1]
    spread = jnp.arange(_EPAD - _E, dtype=jnp.int32) % (_NPAD - _N)
    src_mp = jnp.concatenate([src, spread]).reshape(_NW * _CPW, _CHUNK)
    src_dg = jnp.concatenate([src, _N + spread]).reshape(_NW * _CPW, _CHUNK)
    dst_m = jnp.concatenate([dst, _N + spread]).reshape(_NW * _CPW, _CHUNK)
    srcb = jnp.concatenate([2 * src_mp, 2 * src_mp + 1], axis=0)
    em = jnp.concatenate([src_dg, dst_m], axis=0)
    zn = jnp.zeros((_NPAD,), jnp.float32)
    znd = jnp.zeros((_NPAD, _DH), jnp.bfloat16)

    degs = _deg_kernel(em, zn)
    deg_out = degs[:_N].reshape(_N, 1)
    deg_in = degs[_NPAD:_NPAD + _N].reshape(_N, 1)

    hs1 = _tc1(features, W1, deg_out).reshape(2 * _N, _DH)
    parts1 = _mp_kernel(hs1, srcb, dst_m, znd)

    hs2 = _tc2(parts1[:_N], deg_in, b1.reshape(1, _D), W2,
               deg_out).reshape(2 * _N, _DH)
    parts2 = _mp_kernel(hs2, srcb, dst_m, znd)

    return _tc3(parts2[:_N], deg_in, b2.reshape(1, _D))
